# baseline (speedup 1.0000x reference)
import functools

import jax
import jax.numpy as jnp
from jax import lax
from jax.experimental import pallas as pl
from jax.experimental.pallas import tpu as pltpu
from jax.experimental.pallas import tpu_sc as plsc

E = 8
D = 1024
F = 4096
N = 4096
EPS = 1e-12

BT = 192
FFC = 2048
NJ = F // FFC
PAD_N = N + E * BT
T = PAD_N // BT

NC = 2
NS = 16
NW = NC * NS
TPW = N // NW
CH = 64
CHUNKS = TPW // CH


def _routing(task_ids):
    ids = (task_ids % E).astype(jnp.int32)
    onehot = (ids[:, None] == jnp.arange(E, dtype=jnp.int32)[None, :]).astype(jnp.int32)
    cnt = jnp.sum(onehot, axis=0)
    rank = jnp.take_along_axis(jnp.cumsum(onehot, axis=0), ids[:, None], axis=1)[:, 0] - 1
    region = ((cnt + BT - 1) // BT) * BT
    ends = jnp.cumsum(region)
    starts = ends - region
    dest = (starts[ids] + rank).astype(jnp.int32)

    tile_row = jnp.arange(T, dtype=jnp.int32) * BT
    eot = jnp.clip(jnp.searchsorted(ends, tile_row, side="right"), 0, E - 1).astype(jnp.int32)
    total = ends[-1]
    act = (tile_row < total).astype(jnp.int32)
    n_act = total // BT
    eot_last = jnp.take(eot, n_act - 1)
    eot = jnp.where(act == 1, eot, eot_last)
    meta = jnp.stack([eot, act]).astype(jnp.int32)
    return dest, meta


def _sc_mesh():
    return plsc.VectorSubcoreMesh(core_axis_name="c", subcore_axis_name="s")


def _dispatch(x, dest):
    def body(x_hbm, dest_hbm, xs_hbm, idx_v, rows_v, sem):
        wid = lax.axis_index("s") * NC + lax.axis_index("c")
        for c in range(CHUNKS):
            base = wid * TPW + c * CH
            pltpu.sync_copy(dest_hbm.at[pl.ds(base, CH)], idx_v)
            pltpu.sync_copy(x_hbm.at[pl.ds(base, CH)], rows_v)
            pltpu.async_copy(rows_v, xs_hbm.at[idx_v], sem).wait()

    run = pl.kernel(
        body,
        out_type=jax.ShapeDtypeStruct((PAD_N, D), jnp.float32),
        mesh=_sc_mesh(),
        scratch_types=[
            pltpu.VMEM((CH,), jnp.int32),
            pltpu.VMEM((CH, D), jnp.float32),
            pltpu.SemaphoreType.DMA,
        ],
    )
    return run(x, dest)


def _combine(ys, dest):
    def body(ys_hbm, dest_hbm, out_hbm, idx_v, rows_v, sem):
        wid = lax.axis_index("s") * NC + lax.axis_index("c")
        for c in range(CHUNKS):
            base = wid * TPW + c * CH
            pltpu.sync_copy(dest_hbm.at[pl.ds(base, CH)], idx_v)
            pltpu.async_copy(ys_hbm.at[idx_v], rows_v, sem).wait()
            pltpu.sync_copy(rows_v, out_hbm.at[pl.ds(base, CH)])

    run = pl.kernel(
        body,
        out_type=jax.ShapeDtypeStruct((N, D), jnp.float32),
        mesh=_sc_mesh(),
        scratch_types=[
            pltpu.VMEM((CH,), jnp.int32),
            pltpu.VMEM((CH, D), jnp.float32),
            pltpu.SemaphoreType.DMA,
        ],
    )
    return run(ys, dest)


def _ffn(xs, W1, b1, W2, b2, gamma, beta, meta):
    b1r = b1.reshape(E, 1, F)
    b2r = b2.reshape(E, 1, D)
    gr = gamma.reshape(E, 1, D)
    br = beta.reshape(E, 1, D)

    def body(meta_ref, xs_ref, w1_ref, b1_ref, w2_ref, b2_ref, g_ref, bb_ref,
             out_ref, acc_ref):
        j = pl.program_id(0)
        i = pl.program_id(1)

        @pl.when(meta_ref[1, i] == 1)
        def _():
            x = xs_ref[...]
            h = jnp.dot(x, w1_ref[0], preferred_element_type=jnp.float32) + b1_ref[0]
            h = 0.5 * h * (1.0 + lax.erf(h * 0.7071067811865476))
            part = jnp.dot(h.astype(jnp.bfloat16), w2_ref[0].astype(jnp.bfloat16),
                           preferred_element_type=jnp.float32)

            @pl.when(j == 0)
            def _():
                acc_ref[i] = part.astype(jnp.bfloat16)

            @pl.when(jnp.logical_and(j > 0, j < NJ - 1))
            def _():
                acc_ref[i] = (acc_ref[i].astype(jnp.float32)
                              + part).astype(jnp.bfloat16)

            @pl.when(j == NJ - 1)
            def _():
                z = acc_ref[i].astype(jnp.float32) + part + b2_ref[0] + x
                mu = jnp.mean(z, axis=-1, keepdims=True)
                var = jnp.mean((z - mu) * (z - mu), axis=-1, keepdims=True)
                out_ref[...] = (z - mu) * lax.rsqrt(var + EPS) * g_ref[0] + bb_ref[0]

    grid_spec = pltpu.PrefetchScalarGridSpec(
        num_scalar_prefetch=1,
        grid=(NJ, T),
        in_specs=[
            pl.BlockSpec((BT, D), lambda j, i, m: (i, 0)),
            pl.BlockSpec((1, D, FFC), lambda j, i, m: (m[0, i], 0, j)),
            pl.BlockSpec((1, 1, FFC), lambda j, i, m: (m[0, i], 0, j)),
            pl.BlockSpec((1, FFC, D), lambda j, i, m: (m[0, i], j, 0)),
            pl.BlockSpec((1, 1, D), lambda j, i, m: (m[0, i], 0, 0)),
            pl.BlockSpec((1, 1, D), lambda j, i, m: (m[0, i], 0, 0)),
            pl.BlockSpec((1, 1, D), lambda j, i, m: (m[0, i], 0, 0)),
        ],
        out_specs=pl.BlockSpec(
            (BT, D), lambda j, i, m: (jnp.where(j == NJ - 1, i, 0), 0)),
        scratch_shapes=[pltpu.VMEM((T, BT, D), jnp.bfloat16)],
    )
    return pl.pallas_call(
        body,
        grid_spec=grid_spec,
        out_shape=jax.ShapeDtypeStruct((PAD_N, D), jnp.float32),
    )(meta, xs, W1, b1r, W2, b2r, gr, br)


def kernel(x, task_ids, W1, b1, W2, b2, gamma, beta):
    dest, meta = _routing(task_ids)
    xs = _dispatch(x, dest)
    ys = _ffn(xs, W1, b1, W2, b2, gamma, beta, meta)
    return _combine(ys, dest)


# speedup vs baseline: 1.0010x; 1.0010x over previous
import jax
import jax.numpy as jnp
from jax import lax
from jax.experimental import pallas as pl
from jax.experimental.pallas import tpu as pltpu
from jax.experimental.pallas import tpu_sc as plsc

E = 8
D = 1024
F = 4096
N = 4096
EPS = 1e-12

BT = 192
FFC = 2048
NJ = F // FFC
PAD_N = N + E * BT
T = PAD_N // BT

NC = 2
NS = 16
NW = NC * NS
TPW = N // NW
CH = 64
CHUNKS = TPW // CH


def _routing(task_ids):
    ids = (task_ids % E).astype(jnp.int32)
    onehot = (ids[:, None] == jnp.arange(E, dtype=jnp.int32)[None, :]).astype(jnp.int32)
    cnt = jnp.sum(onehot, axis=0)
    rank = jnp.take_along_axis(jnp.cumsum(onehot, axis=0), ids[:, None], axis=1)[:, 0] - 1
    region = ((cnt + BT - 1) // BT) * BT
    ends = jnp.cumsum(region)
    starts = ends - region
    dest = (starts[ids] + rank).astype(jnp.int32)

    tile_row = jnp.arange(T, dtype=jnp.int32) * BT
    eot = jnp.clip(jnp.searchsorted(ends, tile_row, side="right"), 0, E - 1).astype(jnp.int32)
    total = ends[-1]
    act = (tile_row < total).astype(jnp.int32)
    n_act = total // BT
    eot_last = jnp.take(eot, n_act - 1)
    eot = jnp.where(act == 1, eot, eot_last)
    meta = jnp.stack([eot, act]).astype(jnp.int32)
    return dest, meta


def _sc_mesh():
    return plsc.VectorSubcoreMesh(core_axis_name="c", subcore_axis_name="s")


def _dispatch(x, dest):
    def body(x_hbm, dest_hbm, xs_hbm, idx_v, rows_v, sem):
        wid = lax.axis_index("s") * NC + lax.axis_index("c")
        for c in range(CHUNKS):
            base = wid * TPW + c * CH
            pltpu.sync_copy(dest_hbm.at[pl.ds(base, CH)], idx_v)
            pltpu.sync_copy(x_hbm.at[pl.ds(base, CH)], rows_v)
            pltpu.async_copy(rows_v, xs_hbm.at[idx_v], sem).wait()

    run = pl.kernel(
        body,
        out_type=jax.ShapeDtypeStruct((PAD_N, D), jnp.float32),
        mesh=_sc_mesh(),
        scratch_types=[
            pltpu.VMEM((CH,), jnp.int32),
            pltpu.VMEM((CH, D), jnp.float32),
            pltpu.SemaphoreType.DMA,
        ],
    )
    return run(x, dest)


def _combine(ys, dest):
    def body(ys_hbm, dest_hbm, out_hbm, idx_v, rows_v, sem):
        wid = lax.axis_index("s") * NC + lax.axis_index("c")
        for c in range(CHUNKS):
            base = wid * TPW + c * CH
            pltpu.sync_copy(dest_hbm.at[pl.ds(base, CH)], idx_v)
            pltpu.async_copy(ys_hbm.at[idx_v], rows_v, sem).wait()
            pltpu.sync_copy(rows_v, out_hbm.at[pl.ds(base, CH)])

    run = pl.kernel(
        body,
        out_type=jax.ShapeDtypeStruct((N, D), jnp.float32),
        mesh=_sc_mesh(),
        scratch_types=[
            pltpu.VMEM((CH,), jnp.int32),
            pltpu.VMEM((CH, D), jnp.float32),
            pltpu.SemaphoreType.DMA,
        ],
    )
    return run(ys, dest)


def _ffn(xs, W1, b1, W2, b2, gamma, beta, meta):
    b1r = b1.reshape(E, 1, F)
    b2r = b2.reshape(E, 1, D)
    gr = gamma.reshape(E, 1, D)
    br = beta.reshape(E, 1, D)

    def body(meta_ref, xs_ref, w1_ref, b1_ref, w2_ref, b2_ref, g_ref, bb_ref,
             out_ref, acc_ref):
        j = pl.program_id(0)
        i = pl.program_id(1)

        @pl.when(meta_ref[1, i] == 1)
        def _():
            x = xs_ref[...]
            h = jnp.dot(x.astype(jnp.bfloat16), w1_ref[0].astype(jnp.bfloat16),
                        preferred_element_type=jnp.float32) + b1_ref[0]
            h = 0.5 * h * (1.0 + lax.erf(h * 0.7071067811865476))
            part = jnp.dot(h.astype(jnp.bfloat16), w2_ref[0].astype(jnp.bfloat16),
                           preferred_element_type=jnp.float32)

            @pl.when(j == 0)
            def _():
                acc_ref[i] = part.astype(jnp.bfloat16)

            @pl.when(jnp.logical_and(j > 0, j < NJ - 1))
            def _():
                acc_ref[i] = (acc_ref[i].astype(jnp.float32)
                              + part).astype(jnp.bfloat16)

            @pl.when(j == NJ - 1)
            def _():
                z = acc_ref[i].astype(jnp.float32) + part + b2_ref[0] + x
                mu = jnp.mean(z, axis=-1, keepdims=True)
                var = jnp.mean((z - mu) * (z - mu), axis=-1, keepdims=True)
                out_ref[...] = (z - mu) * lax.rsqrt(var + EPS) * g_ref[0] + bb_ref[0]

    grid_spec = pltpu.PrefetchScalarGridSpec(
        num_scalar_prefetch=1,
        grid=(NJ, T),
        in_specs=[
            pl.BlockSpec((BT, D), lambda j, i, m: (i, 0)),
            pl.BlockSpec((1, D, FFC), lambda j, i, m: (m[0, i], 0, j)),
            pl.BlockSpec((1, 1, FFC), lambda j, i, m: (m[0, i], 0, j)),
            pl.BlockSpec((1, FFC, D), lambda j, i, m: (m[0, i], j, 0)),
            pl.BlockSpec((1, 1, D), lambda j, i, m: (m[0, i], 0, 0)),
            pl.BlockSpec((1, 1, D), lambda j, i, m: (m[0, i], 0, 0)),
            pl.BlockSpec((1, 1, D), lambda j, i, m: (m[0, i], 0, 0)),
        ],
        out_specs=pl.BlockSpec(
            (BT, D), lambda j, i, m: (jnp.where(j == NJ - 1, i, 0), 0)),
        scratch_shapes=[pltpu.VMEM((T, BT, D), jnp.bfloat16)],
    )
    return pl.pallas_call(
        body,
        grid_spec=grid_spec,
        out_shape=jax.ShapeDtypeStruct((PAD_N, D), jnp.float32),
    )(meta, xs, W1, b1r, W2, b2r, gr, br)


def kernel(x, task_ids, W1, b1, W2, b2, gamma, beta):
    dest, meta = _routing(task_ids)
    xs = _dispatch(x, dest)
    ys = _ffn(xs, W1, b1, W2, b2, gamma, beta, meta)
    return _combine(ys, dest)
